# baseline (speedup 1.0000x reference)
import functools

import jax
import jax.numpy as jnp
from jax import lax
from jax.experimental import pallas as pl
from jax.experimental.pallas import tpu as pltpu
from jax.experimental.pallas import tpu_sc as plsc

N = 10000
E = 320000
D_IN = 128
HID = 16
OUT = 2
G = 64

NC = 2
NS = 16
NW = NC * NS
NB = 6
LA = 3
CHUNK = 1000
NITER = 10
E_W = NITER * CHUNK
NP = 10112
NPAD = NP - N
STRIPE = NP // NS

@functools.lru_cache(maxsize=1)
def _sc_mesh():
    return plsc.VectorSubcoreMesh(
        core_axis_name="c", subcore_axis_name="s", num_cores=NC, num_subcores=NS)


def _deg_body(edge_hbm, zeros_hbm, ones_hbm, out_hbm, dstv, rows, accum, sem):
    c = lax.axis_index("c")
    s = lax.axis_index("s")
    wid = c * NS + s
    row0 = s * STRIPE
    base = wid * E_W
    for t in range(NITER):
        pltpu.async_copy(
            edge_hbm.at[1, pl.ds(base + CHUNK * t, CHUNK)], dstv.at[t], sem)
    pltpu.sync_copy(zeros_hbm.at[pl.ds(row0, STRIPE)], accum.at[pl.ds(row0, STRIPE)])
    pltpu.sync_copy(ones_hbm, rows)
    for t in range(NITER):
        pltpu.make_async_copy(
            edge_hbm.at[1, pl.ds(base + CHUNK * t, CHUNK)], dstv.at[t], sem).wait()
    plsc.subcore_barrier()
    for t in range(NITER):
        pltpu.async_copy(rows, accum.at[dstv.at[t]], sem, add=True)
    for t in range(NITER):
        pltpu.make_async_copy(rows, accum.at[dstv.at[t]], sem).wait()
    plsc.subcore_barrier()
    pltpu.sync_copy(accum.at[pl.ds(row0, STRIPE)], out_hbm.at[c, pl.ds(row0, STRIPE)])


@functools.lru_cache(maxsize=1)
def _deg_call():
    return pl.kernel(
        _deg_body,
        out_type=jax.ShapeDtypeStruct((NC, NP, HID), jnp.float32),
        mesh=_sc_mesh(),
        scratch_types=[
            pltpu.VMEM((NITER, CHUNK), jnp.int32),
            pltpu.VMEM((CHUNK, HID), jnp.float32),
            pltpu.VMEM_SHARED((NP, HID), jnp.float32),
            pltpu.SemaphoreType.DMA,
        ],
        compiler_params=pltpu.CompilerParams(use_tc_tiling_on_sc=False),
    )


def _agg_body(g_hbm, edge_hbm, zeros_hbm, out_hbm, srcv, dstv, rows, accum, gsem, ssem):
    c = lax.axis_index("c")
    s = lax.axis_index("s")
    wid = c * NS + s
    row0 = s * STRIPE
    base = wid * E_W
    for t in range(NITER):
        pltpu.async_copy(
            edge_hbm.at[0, pl.ds(base + CHUNK * t, CHUNK)], srcv.at[t], ssem.at[0])
        pltpu.async_copy(
            edge_hbm.at[1, pl.ds(base + CHUNK * t, CHUNK)], dstv.at[t], ssem.at[1])
    pltpu.sync_copy(zeros_hbm.at[pl.ds(row0, STRIPE)], accum.at[pl.ds(row0, STRIPE)])
    for t in range(NITER):
        pltpu.make_async_copy(
            edge_hbm.at[0, pl.ds(base + CHUNK * t, CHUNK)], srcv.at[t], ssem.at[0]).wait()
        pltpu.make_async_copy(
            edge_hbm.at[1, pl.ds(base + CHUNK * t, CHUNK)], dstv.at[t], ssem.at[1]).wait()
    plsc.subcore_barrier()

    for t in range(min(LA, NITER)):
        pltpu.async_copy(g_hbm.at[srcv.at[t]], rows.at[t % NB], gsem.at[t % NB])
    for t in range(NITER):
        b = t % NB
        pltpu.make_async_copy(g_hbm.at[srcv.at[t]], rows.at[b], gsem.at[b]).wait()
        pltpu.async_copy(rows.at[b], accum.at[dstv.at[t]], ssem.at[b], add=True)
        tn = t + LA
        if tn < NITER:
            bn = tn % NB
            if tn >= NB:
                tp = tn - NB
                pltpu.make_async_copy(
                    rows.at[bn], accum.at[dstv.at[tp]], ssem.at[bn]).wait()
            pltpu.async_copy(g_hbm.at[srcv.at[tn]], rows.at[bn], gsem.at[bn])
    for t in range(NITER - NB, NITER):
        b = t % NB
        pltpu.make_async_copy(rows.at[b], accum.at[dstv.at[t]], ssem.at[b]).wait()
    plsc.subcore_barrier()
    pltpu.sync_copy(accum.at[pl.ds(row0, STRIPE)], out_hbm.at[c, pl.ds(row0, STRIPE)])


@functools.lru_cache(maxsize=1)
def _agg_call():
    return pl.kernel(
        _agg_body,
        out_type=jax.ShapeDtypeStruct((NC, NP, HID), jnp.float32),
        mesh=_sc_mesh(),
        scratch_types=[
            pltpu.VMEM((NITER, CHUNK), jnp.int32),
            pltpu.VMEM((NITER, CHUNK), jnp.int32),
            pltpu.VMEM((NB, CHUNK, HID), jnp.float32),
            pltpu.VMEM_SHARED((NP, HID), jnp.float32),
            pltpu.SemaphoreType.DMA((NB,)),
            pltpu.SemaphoreType.DMA((NB,)),
        ],
        compiler_params=pltpu.CompilerParams(use_tc_tiling_on_sc=False),
    )


NPR = NP // 8
NR = N // 8


def _dinv_packed(deg2):
    deg = deg2[0] + deg2[1] + 1.0
    dinv = lax.rsqrt(deg)
    rows = lax.broadcasted_iota(jnp.int32, (NPR, 128), 0)
    return jnp.where(rows < NR, dinv, 0.0)


def _tc_first_body(deg2_ref, x2_ref, w_ref, g_ref, dinv_ref):
    dinv = _dinv_packed(deg2_ref[...])
    dinv_ref[...] = dinv
    zp = jnp.dot(x2_ref[...], w_ref[...], preferred_element_type=jnp.float32)
    g_ref[...] = dinv * zp


def _tc_first(deg2p, x2, w0big):
    return pl.pallas_call(
        _tc_first_body,
        out_shape=(
            jax.ShapeDtypeStruct((NPR, 128), jnp.float32),
            jax.ShapeDtypeStruct((NPR, 128), jnp.float32),
        ),
    )(deg2p, x2, w0big)


def _tc_mid_body(s2_ref, g_ref, dinv_ref, b_ref, a_ref, w_ref, out_ref):
    dinv = dinv_ref[...]
    pre = dinv * (s2_ref[0] + s2_ref[1] + g_ref[...]) + b_ref[...]
    a = a_ref[0, 0]
    h = jnp.where(pre >= 0.0, pre, a * pre)
    z = jnp.dot(h, w_ref[...], preferred_element_type=jnp.float32)
    out_ref[...] = dinv * z


def _tc_mid(s2p, gp, dinv_p, bt, a, wbig):
    return pl.pallas_call(
        _tc_mid_body,
        out_shape=jax.ShapeDtypeStruct((NPR, 128), jnp.float32),
    )(s2p, gp, dinv_p, bt, a, wbig)


def _tc_pool_body(s2_ref, g_ref, dinv_ref, b_ref, batch_ref, out_ref):
    hn = dinv_ref[...] * (s2_ref[0] + s2_ref[1] + g_ref[...])
    gid = lax.broadcasted_iota(jnp.int32, (G, NPR), 0)
    sums = jnp.zeros((G, HID), jnp.float32)
    cnt = jnp.zeros((G, HID), jnp.float32)
    ones = jnp.ones((NPR, HID), jnp.float32)
    for a in range(8):
        oh = (batch_ref[a:a + 1, :] == gid).astype(jnp.float32)
        sums = sums + jnp.dot(oh, hn[:, 16 * a:16 * (a + 1)],
                              preferred_element_type=jnp.float32)
        cnt = cnt + jnp.dot(oh, ones, preferred_element_type=jnp.float32)
    out_ref[...] = jnp.where(
        cnt >= 0.5, sums / jnp.maximum(cnt, 1.0) + b_ref[...], 0.0)


def _tc_pool(s2p, gp, dinv_p, b3p, batch8):
    return pl.pallas_call(
        _tc_pool_body,
        out_shape=jax.ShapeDtypeStruct((G, HID), jnp.float32),
    )(s2p, gp, dinv_p, b3p, batch8)


def _packed(arr2d):
    return arr2d.reshape(arr2d.shape[:-2] + (arr2d.shape[-2] // 8, 128))


def _flat(arrp):
    return arrp.reshape(arrp.shape[:-2] + (arrp.shape[-2] * 8, HID))


def kernel(x, edge_index, batch, W0, b0, a0, W1, b1, a1, W2, b2, a2, W3, b3):
    zeros_tab = jnp.zeros((NP, HID), jnp.float32)
    ones_rows = jnp.ones((CHUNK, HID), jnp.float32)
    eye8 = jnp.eye(8, dtype=jnp.float32)
    x2 = jnp.pad(x, ((0, NPAD), (0, 0))).reshape(NPR, 8 * D_IN)
    w0big = jnp.kron(eye8, W0)
    w1big = jnp.kron(eye8, W1)
    w2big = jnp.kron(eye8, W2)
    w3big = jnp.kron(eye8, jnp.pad(W3, ((0, 0), (0, HID - OUT))))
    b0t = jnp.tile(b0, 8).reshape(1, 128)
    b1t = jnp.tile(b1, 8).reshape(1, 128)
    b2t = jnp.tile(b2, 8).reshape(1, 128)
    b3p = jnp.pad(b3, (0, HID - OUT)).reshape(1, HID)
    a0r = a0.reshape(1, 1)
    a1r = a1.reshape(1, 1)
    a2r = a2.reshape(1, 1)
    batch8 = jnp.pad(batch, (0, NPAD), constant_values=-1).reshape(NPR, 8).T

    deg2 = _deg_call()(edge_index, zeros_tab, ones_rows)
    g0p, dinv_p = _tc_first(_packed(deg2), x2, w0big)
    s0 = _agg_call()(_flat(g0p), edge_index, zeros_tab)
    g1p = _tc_mid(_packed(s0), g0p, dinv_p, b0t, a0r, w1big)
    s1 = _agg_call()(_flat(g1p), edge_index, zeros_tab)
    g2p = _tc_mid(_packed(s1), g1p, dinv_p, b1t, a1r, w2big)
    s2 = _agg_call()(_flat(g2p), edge_index, zeros_tab)
    g3p = _tc_mid(_packed(s2), g2p, dinv_p, b2t, a2r, w3big)
    s3 = _agg_call()(_flat(g3p), edge_index, zeros_tab)
    out16 = _tc_pool(_packed(s3), g3p, dinv_p, b3p, batch8)
    return out16[:, :OUT]


# speedup vs baseline: 1.0355x; 1.0355x over previous
import functools

import jax
import jax.numpy as jnp
from jax import lax
from jax.experimental import pallas as pl
from jax.experimental.pallas import tpu as pltpu
from jax.experimental.pallas import tpu_sc as plsc

N = 10000
E = 320000
D_IN = 128
HID = 16
OUT = 2
G = 64

NC = 2
NS = 16
NW = NC * NS
NB = 4
LA = 2
CHUNK = 1000
NITER = 10
E_W = NITER * CHUNK
NP = 10112
NPAD = NP - N
STRIPE = NP // NS

@functools.lru_cache(maxsize=1)
def _sc_mesh():
    return plsc.VectorSubcoreMesh(
        core_axis_name="c", subcore_axis_name="s", num_cores=NC, num_subcores=NS)


def _deg_body(edge_hbm, zeros_hbm, ones_hbm, out_hbm, dstv, rows, accum, sem):
    c = lax.axis_index("c")
    s = lax.axis_index("s")
    wid = c * NS + s
    row0 = s * STRIPE
    base = wid * E_W
    for t in range(NITER):
        pltpu.async_copy(
            edge_hbm.at[1, pl.ds(base + CHUNK * t, CHUNK)], dstv.at[t], sem)
    pltpu.sync_copy(zeros_hbm.at[pl.ds(row0, STRIPE)], accum.at[pl.ds(row0, STRIPE)])
    pltpu.sync_copy(ones_hbm, rows)
    for t in range(NITER):
        pltpu.make_async_copy(
            edge_hbm.at[1, pl.ds(base + CHUNK * t, CHUNK)], dstv.at[t], sem).wait()
    plsc.subcore_barrier()
    for t in range(NITER):
        pltpu.async_copy(rows, accum.at[dstv.at[t]], sem, add=True)
    for t in range(NITER):
        pltpu.make_async_copy(rows, accum.at[dstv.at[t]], sem).wait()
    plsc.subcore_barrier()
    pltpu.sync_copy(accum.at[pl.ds(row0, STRIPE)], out_hbm.at[c, pl.ds(row0, STRIPE)])


@functools.lru_cache(maxsize=1)
def _deg_call():
    return pl.kernel(
        _deg_body,
        out_type=jax.ShapeDtypeStruct((NC, NP, HID), jnp.float32),
        mesh=_sc_mesh(),
        scratch_types=[
            pltpu.VMEM((NITER, CHUNK), jnp.int32),
            pltpu.VMEM((CHUNK, HID), jnp.float32),
            pltpu.VMEM_SHARED((NP, HID), jnp.float32),
            pltpu.SemaphoreType.DMA,
        ],
        compiler_params=pltpu.CompilerParams(use_tc_tiling_on_sc=False),
    )


def _agg_body(g_hbm, edge_hbm, zeros_hbm, out_hbm, srcv, dstv, rows, accum, gsem, ssem):
    c = lax.axis_index("c")
    s = lax.axis_index("s")
    wid = c * NS + s
    row0 = s * STRIPE
    base = wid * E_W
    for t in range(NITER):
        pltpu.async_copy(
            edge_hbm.at[0, pl.ds(base + CHUNK * t, CHUNK)], srcv.at[t], ssem.at[0])
        pltpu.async_copy(
            edge_hbm.at[1, pl.ds(base + CHUNK * t, CHUNK)], dstv.at[t], ssem.at[1])
    pltpu.sync_copy(zeros_hbm.at[pl.ds(row0, STRIPE)], accum.at[pl.ds(row0, STRIPE)])
    for t in range(NITER):
        pltpu.make_async_copy(
            edge_hbm.at[0, pl.ds(base + CHUNK * t, CHUNK)], srcv.at[t], ssem.at[0]).wait()
        pltpu.make_async_copy(
            edge_hbm.at[1, pl.ds(base + CHUNK * t, CHUNK)], dstv.at[t], ssem.at[1]).wait()
    plsc.subcore_barrier()

    for t in range(min(LA, NITER)):
        pltpu.async_copy(g_hbm.at[srcv.at[t]], rows.at[t % NB], gsem.at[t % NB])
    for t in range(NITER):
        b = t % NB
        pltpu.make_async_copy(g_hbm.at[srcv.at[t]], rows.at[b], gsem.at[b]).wait()
        pltpu.async_copy(rows.at[b], accum.at[dstv.at[t]], ssem.at[b], add=True)
        tn = t + LA
        if tn < NITER:
            bn = tn % NB
            if tn >= NB:
                tp = tn - NB
                pltpu.make_async_copy(
                    rows.at[bn], accum.at[dstv.at[tp]], ssem.at[bn]).wait()
            pltpu.async_copy(g_hbm.at[srcv.at[tn]], rows.at[bn], gsem.at[bn])
    for t in range(NITER - NB, NITER):
        b = t % NB
        pltpu.make_async_copy(rows.at[b], accum.at[dstv.at[t]], ssem.at[b]).wait()
    plsc.subcore_barrier()
    pltpu.sync_copy(accum.at[pl.ds(row0, STRIPE)], out_hbm.at[c, pl.ds(row0, STRIPE)])


@functools.lru_cache(maxsize=1)
def _agg_call():
    return pl.kernel(
        _agg_body,
        out_type=jax.ShapeDtypeStruct((NC, NP, HID), jnp.float32),
        mesh=_sc_mesh(),
        scratch_types=[
            pltpu.VMEM((NITER, CHUNK), jnp.int32),
            pltpu.VMEM((NITER, CHUNK), jnp.int32),
            pltpu.VMEM((NB, CHUNK, HID), jnp.float32),
            pltpu.VMEM_SHARED((NP, HID), jnp.float32),
            pltpu.SemaphoreType.DMA((NB,)),
            pltpu.SemaphoreType.DMA((NB,)),
        ],
        compiler_params=pltpu.CompilerParams(use_tc_tiling_on_sc=False),
    )


NPR = NP // 8
NR = N // 8


def _dinv_packed(deg2):
    deg = deg2[0] + deg2[1] + 1.0
    dinv = lax.rsqrt(deg)
    rows = lax.broadcasted_iota(jnp.int32, (NPR, 128), 0)
    return jnp.where(rows < NR, dinv, 0.0)


def _tc_z0_body(x2_ref, w_ref, z_ref):
    z_ref[...] = jnp.dot(x2_ref[...], w_ref[...], preferred_element_type=jnp.float32)


def _tc_z0(x2, w0big):
    return pl.pallas_call(
        _tc_z0_body,
        out_shape=jax.ShapeDtypeStruct((NPR, 128), jnp.float32),
    )(x2, w0big)


def _tc_first_body(deg2_ref, z_ref, g_ref, dinv_ref):
    dinv = _dinv_packed(deg2_ref[...])
    dinv_ref[...] = dinv
    g_ref[...] = dinv * z_ref[...]


def _tc_first(deg2p, z0p):
    return pl.pallas_call(
        _tc_first_body,
        out_shape=(
            jax.ShapeDtypeStruct((NPR, 128), jnp.float32),
            jax.ShapeDtypeStruct((NPR, 128), jnp.float32),
        ),
    )(deg2p, z0p)


def _tc_mid_body(s2_ref, g_ref, dinv_ref, b_ref, a_ref, w_ref, out_ref):
    dinv = dinv_ref[...]
    pre = dinv * (s2_ref[0] + s2_ref[1] + g_ref[...]) + b_ref[...]
    a = a_ref[0, 0]
    h = jnp.where(pre >= 0.0, pre, a * pre)
    z = jnp.dot(h, w_ref[...], preferred_element_type=jnp.float32)
    out_ref[...] = dinv * z


def _tc_mid(s2p, gp, dinv_p, bt, a, wbig):
    return pl.pallas_call(
        _tc_mid_body,
        out_shape=jax.ShapeDtypeStruct((NPR, 128), jnp.float32),
    )(s2p, gp, dinv_p, bt, a, wbig)


def _tc_pool_body(s2_ref, g_ref, dinv_ref, b_ref, batch_ref, out_ref):
    hn = dinv_ref[...] * (s2_ref[0] + s2_ref[1] + g_ref[...])
    gid = lax.broadcasted_iota(jnp.int32, (G, NPR), 0)
    sums = jnp.zeros((G, HID), jnp.float32)
    cnt = jnp.zeros((G, HID), jnp.float32)
    ones = jnp.ones((NPR, HID), jnp.float32)
    for a in range(8):
        oh = (batch_ref[a:a + 1, :] == gid).astype(jnp.float32)
        sums = sums + jnp.dot(oh, hn[:, 16 * a:16 * (a + 1)],
                              preferred_element_type=jnp.float32)
        cnt = cnt + jnp.dot(oh, ones, preferred_element_type=jnp.float32)
    out_ref[...] = jnp.where(
        cnt >= 0.5, sums / jnp.maximum(cnt, 1.0) + b_ref[...], 0.0)


def _tc_pool(s2p, gp, dinv_p, b3p, batch8):
    return pl.pallas_call(
        _tc_pool_body,
        out_shape=jax.ShapeDtypeStruct((G, HID), jnp.float32),
    )(s2p, gp, dinv_p, b3p, batch8)


def _packed(arr2d):
    return arr2d.reshape(arr2d.shape[:-2] + (arr2d.shape[-2] // 8, 128))


def _flat(arrp):
    return arrp.reshape(arrp.shape[:-2] + (arrp.shape[-2] * 8, HID))


def kernel(x, edge_index, batch, W0, b0, a0, W1, b1, a1, W2, b2, a2, W3, b3):
    zeros_tab = jnp.zeros((NP, HID), jnp.float32)
    ones_rows = jnp.ones((CHUNK, HID), jnp.float32)
    eye8 = jnp.eye(8, dtype=jnp.float32)
    x2 = jnp.pad(x, ((0, NPAD), (0, 0))).reshape(NPR, 8 * D_IN)
    w0big = jnp.kron(eye8, W0)
    w1big = jnp.kron(eye8, W1)
    w2big = jnp.kron(eye8, W2)
    w3big = jnp.kron(eye8, jnp.pad(W3, ((0, 0), (0, HID - OUT))))
    b0t = jnp.tile(b0, 8).reshape(1, 128)
    b1t = jnp.tile(b1, 8).reshape(1, 128)
    b2t = jnp.tile(b2, 8).reshape(1, 128)
    b3p = jnp.pad(b3, (0, HID - OUT)).reshape(1, HID)
    a0r = a0.reshape(1, 1)
    a1r = a1.reshape(1, 1)
    a2r = a2.reshape(1, 1)
    batch8 = jnp.pad(batch, (0, NPAD), constant_values=-1).reshape(NPR, 8).T

    z0p = _tc_z0(x2, w0big)
    deg2 = _deg_call()(edge_index, zeros_tab, ones_rows)
    g0p, dinv_p = _tc_first(_packed(deg2), z0p)
    s0 = _agg_call()(_flat(g0p), edge_index, zeros_tab)
    g1p = _tc_mid(_packed(s0), g0p, dinv_p, b0t, a0r, w1big)
    s1 = _agg_call()(_flat(g1p), edge_index, zeros_tab)
    g2p = _tc_mid(_packed(s1), g1p, dinv_p, b1t, a1r, w2big)
    s2 = _agg_call()(_flat(g2p), edge_index, zeros_tab)
    g3p = _tc_mid(_packed(s2), g2p, dinv_p, b2t, a2r, w3big)
    s3 = _agg_call()(_flat(g3p), edge_index, zeros_tab)
    out16 = _tc_pool(_packed(s3), g3p, dinv_p, b3p, batch8)
    return out16[:, :OUT]
